# baseline (speedup 1.0000x reference)
import functools

import jax
import jax.numpy as jnp
from jax import lax
from jax.experimental import pallas as pl
from jax.experimental.pallas import tpu as pltpu
from jax.experimental.pallas import tpu_sc as plsc

VOCAB = 256
D = 64
B = 4096
N = 200

NUM_CORES = 2
NUM_SUBCORES = 16
NW = NUM_CORES * NUM_SUBCORES

TOTAL = B * N
PER_W = TOTAL // NW
CHUNK = 512
NCHUNK = PER_W // CHUNK


def _scale_body(t_ref, o_ref):
    o_ref[...] = t_ref[...] * (D ** 0.5)


def _scale_table(table):
    return pl.pallas_call(
        _scale_body,
        out_shape=jax.ShapeDtypeStruct((VOCAB, D), jnp.float32),
    )(table)


def _sc_body(table_hbm, idx_hbm, out_hbm, table_v, idx_v, rows_v, gsem, ssem):
    wid = lax.axis_index("s") * NUM_CORES + lax.axis_index("c")
    base = wid * PER_W
    sid = lax.axis_index("s")

    @pl.when(sid == 0)
    def _():
        pltpu.sync_copy(table_hbm, table_v)

    plsc.subcore_barrier()
    pltpu.sync_copy(idx_hbm.at[wid], idx_v)

    def start_gather(j, p):
        pltpu.async_copy(table_v.at[idx_v.at[j]], rows_v.at[p], gsem.at[p])

    def wait_gather(j, p):
        pltpu.make_async_copy(
            table_v.at[idx_v.at[j]], rows_v.at[p], gsem.at[p]
        ).wait()

    def start_scatter(j, p):
        pltpu.async_copy(
            rows_v.at[p], out_hbm.at[pl.ds(base + j * CHUNK, CHUNK)], ssem.at[p]
        )

    def wait_scatter(p):
        pltpu.make_async_copy(
            rows_v.at[p], out_hbm.at[pl.ds(base, CHUNK)], ssem.at[p]
        ).wait()

    start_gather(0, 0)

    def step(j, carry):
        p = lax.rem(j, 2)
        q = 1 - p
        nj = j + 1

        @pl.when(nj < NCHUNK)
        def _():
            @pl.when(nj >= 2)
            def _():
                wait_scatter(q)

            start_gather(nj, q)

        wait_gather(j, p)
        start_scatter(j, p)
        return carry

    lax.fori_loop(0, NCHUNK, step, 0)
    wait_scatter(0)
    wait_scatter(1)


@jax.jit
def kernel(indices, embedding_matrix):
    table = _scale_table(embedding_matrix.astype(jnp.float32))
    idx = indices.astype(jnp.int32).reshape(NW, NCHUNK, CHUNK)

    mesh = plsc.VectorSubcoreMesh(core_axis_name="c", subcore_axis_name="s")
    out = pl.kernel(
        _sc_body,
        out_type=jax.ShapeDtypeStruct((TOTAL, D), jnp.float32),
        mesh=mesh,
        compiler_params=pltpu.CompilerParams(use_tc_tiling_on_sc=False),
        scratch_types=[
            pltpu.VMEM_SHARED((VOCAB, D), jnp.float32),
            pltpu.VMEM((NCHUNK, CHUNK), jnp.int32),
            pltpu.VMEM((2, CHUNK, D), jnp.float32),
            pltpu.SemaphoreType.DMA((2,)),
            pltpu.SemaphoreType.DMA((2,)),
        ],
    )(table, idx)
    return out.reshape(B, N, D)


# speedup vs baseline: 1.0000x; 1.0000x over previous
import jax
import jax.numpy as jnp
from jax import lax
from jax.experimental import pallas as pl
from jax.experimental.pallas import tpu as pltpu
from jax.experimental.pallas import tpu_sc as plsc

VOCAB = 256
D = 64
B = 4096
N = 200

NUM_CORES = 2
NUM_SUBCORES = 16
NW = NUM_CORES * NUM_SUBCORES

NB_PER_W = B // NW


def _scale_body(t_ref, o_ref):
    o_ref[...] = t_ref[...] * (D ** 0.5)


def _scale_table(table):
    return pl.pallas_call(
        _scale_body,
        out_shape=jax.ShapeDtypeStruct((VOCAB, D), jnp.float32),
    )(table)


def _sc_body(table_hbm, idx_hbm, out_hbm, table_v, idx_v, rows_v, gsem, ssem):
    wid = lax.axis_index("s") * NUM_CORES + lax.axis_index("c")
    sid = lax.axis_index("s")
    b0 = wid * NB_PER_W

    @pl.when(sid == 0)
    def _():
        pltpu.sync_copy(table_hbm, table_v)

    plsc.subcore_barrier()
    pltpu.sync_copy(idx_hbm.at[pl.ds(b0, NB_PER_W)], idx_v)

    def start_gather(j, p):
        pltpu.async_copy(table_v.at[idx_v.at[j]], rows_v.at[p], gsem.at[p])

    def wait_gather(j, p):
        pltpu.make_async_copy(
            table_v.at[idx_v.at[j]], rows_v.at[p], gsem.at[p]
        ).wait()

    def start_scatter(j, p):
        pltpu.async_copy(rows_v.at[p], out_hbm.at[b0 + j], ssem.at[p])

    def wait_scatter(p):
        pltpu.make_async_copy(rows_v.at[p], out_hbm.at[b0], ssem.at[p]).wait()

    start_gather(0, 0)

    def step(j, carry):
        p = lax.rem(j, 2)
        q = 1 - p
        nj = j + 1

        @pl.when(nj < NB_PER_W)
        def _():
            @pl.when(nj >= 2)
            def _():
                wait_scatter(q)

            start_gather(nj, q)

        wait_gather(j, p)
        start_scatter(j, p)
        return carry

    lax.fori_loop(0, NB_PER_W, step, 0)
    wait_scatter(0)
    wait_scatter(1)


@jax.jit
def kernel(indices, embedding_matrix):
    table = _scale_table(embedding_matrix.astype(jnp.float32))
    idx = indices.astype(jnp.int32)

    mesh = plsc.VectorSubcoreMesh(core_axis_name="c", subcore_axis_name="s")
    out = pl.kernel(
        _sc_body,
        out_type=jax.ShapeDtypeStruct((B, N, D), jnp.float32),
        mesh=mesh,
        compiler_params=pltpu.CompilerParams(use_tc_tiling_on_sc=False),
        scratch_types=[
            pltpu.VMEM_SHARED((VOCAB, D), jnp.float32),
            pltpu.VMEM((NB_PER_W, N), jnp.int32),
            pltpu.VMEM((2, N, D), jnp.float32),
            pltpu.SemaphoreType.DMA((2,)),
            pltpu.SemaphoreType.DMA((2,)),
        ],
    )(table, idx)
    return out


# speedup vs baseline: 1.3093x; 1.3092x over previous
import jax
import jax.numpy as jnp
from jax import lax
from jax.experimental import pallas as pl
from jax.experimental.pallas import tpu as pltpu
from jax.experimental.pallas import tpu_sc as plsc

VOCAB = 256
D = 64
B = 4096
N = 200

LANES = 128
NPAD = 256
TAIL = N - LANES

NUM_CORES = 2
NUM_SUBCORES = 16
NW = NUM_CORES * NUM_SUBCORES

NB_PER_W = B // NW
NCHUNK = NB_PER_W * (NPAD // LANES)


def _scale_body(t_ref, o_ref):
    o_ref[...] = jnp.concatenate(
        [t_ref[...] * (D ** 0.5), jnp.zeros((VOCAB, LANES - D), jnp.float32)],
        axis=1,
    )


def _scale_pad_table(table):
    return pl.pallas_call(
        _scale_body,
        out_shape=jax.ShapeDtypeStruct((VOCAB, LANES), jnp.float32),
    )(table)


def _sc_body(table_hbm, idx_hbm, out_hbm, table_v, idx_v, rows_v, gsem, ssem):
    wid = lax.axis_index("s") * NUM_CORES + lax.axis_index("c")
    sid = lax.axis_index("s")
    b0 = wid * NB_PER_W

    @pl.when(sid == 0)
    def _():
        pltpu.sync_copy(table_hbm, table_v)

    plsc.subcore_barrier()
    pltpu.sync_copy(idx_hbm.at[pl.ds(wid * NCHUNK, NCHUNK)], idx_v)

    def start_gather(c, p):
        pltpu.async_copy(table_v.at[idx_v.at[c]], rows_v.at[p], gsem.at[p])

    def wait_gather(c, p):
        pltpu.make_async_copy(
            table_v.at[idx_v.at[c]], rows_v.at[p], gsem.at[p]
        ).wait()

    def start_scatter(c, p):
        b = b0 + lax.div(c, 2)

        @pl.when(p == 0)
        def _():
            pltpu.async_copy(
                rows_v.at[0], out_hbm.at[b, pl.ds(0, LANES)], ssem.at[0]
            )

        @pl.when(p == 1)
        def _():
            pltpu.async_copy(
                rows_v.at[1, pl.ds(0, TAIL)],
                out_hbm.at[b, pl.ds(LANES, TAIL)],
                ssem.at[1],
            )

    def wait_scatter(p):
        @pl.when(p == 0)
        def _():
            pltpu.make_async_copy(
                rows_v.at[0], out_hbm.at[b0, pl.ds(0, LANES)], ssem.at[0]
            ).wait()

        @pl.when(p == 1)
        def _():
            pltpu.make_async_copy(
                rows_v.at[1, pl.ds(0, TAIL)],
                out_hbm.at[b0, pl.ds(LANES, TAIL)],
                ssem.at[1],
            ).wait()

    start_gather(0, 0)

    def step(c, carry):
        p = lax.rem(c, 2)
        q = 1 - p
        nc = c + 1

        @pl.when(nc < NCHUNK)
        def _():
            @pl.when(nc >= 2)
            def _():
                wait_scatter(q)

            start_gather(nc, q)

        wait_gather(c, p)
        start_scatter(c, p)
        return carry

    lax.fori_loop(0, NCHUNK, step, 0)
    wait_scatter(0)
    wait_scatter(1)


@jax.jit
def kernel(indices, embedding_matrix):
    table = _scale_pad_table(embedding_matrix.astype(jnp.float32))
    idx = jnp.pad(indices.astype(jnp.int32), ((0, 0), (0, NPAD - N)))
    idx = idx.reshape(NW * NCHUNK, LANES)

    mesh = plsc.VectorSubcoreMesh(core_axis_name="c", subcore_axis_name="s")
    out = pl.kernel(
        _sc_body,
        out_type=jax.ShapeDtypeStruct((B, N, LANES), jnp.float32),
        mesh=mesh,
        scratch_types=[
            pltpu.VMEM_SHARED((VOCAB, LANES), jnp.float32),
            pltpu.VMEM((NCHUNK, LANES), jnp.int32),
            pltpu.VMEM((2, LANES, LANES), jnp.float32),
            pltpu.SemaphoreType.DMA((2,)),
            pltpu.SemaphoreType.DMA((2,)),
        ],
    )(table, idx)
    return out[:, :, :D]


# speedup vs baseline: 1.3710x; 1.0471x over previous
import jax
import jax.numpy as jnp
from jax import lax
from jax.experimental import pallas as pl
from jax.experimental.pallas import tpu as pltpu
from jax.experimental.pallas import tpu_sc as plsc

VOCAB = 256
D = 64
B = 4096
N = 200

LANES = 128
NPAD = 256
TAIL = N - LANES

NUM_CORES = 2
NUM_SUBCORES = 16
NW = NUM_CORES * NUM_SUBCORES

NB_PER_W = B // NW
NCHUNK = NB_PER_W * (NPAD // LANES)
NBUF = 4


def _scale_body(t_ref, o_ref):
    o_ref[...] = jnp.concatenate(
        [t_ref[...] * (D ** 0.5), jnp.zeros((VOCAB, LANES - D), jnp.float32)],
        axis=1,
    )


def _scale_pad_table(table):
    return pl.pallas_call(
        _scale_body,
        out_shape=jax.ShapeDtypeStruct((VOCAB, LANES), jnp.float32),
    )(table)


def _sc_body(table_hbm, idx_hbm, out_hbm, table_v, idx_v, rows_v, gsem, ssem):
    wid = lax.axis_index("s") * NUM_CORES + lax.axis_index("c")
    sid = lax.axis_index("s")
    b0 = wid * NB_PER_W

    @pl.when(sid == 0)
    def _():
        pltpu.sync_copy(table_hbm, table_v)

    plsc.subcore_barrier()
    pltpu.sync_copy(idx_hbm.at[pl.ds(wid * NCHUNK, NCHUNK)], idx_v)

    def start_gather(c, m):
        pltpu.async_copy(table_v.at[idx_v.at[c]], rows_v.at[m], gsem.at[m])

    def wait_gather(c, m):
        pltpu.make_async_copy(
            table_v.at[idx_v.at[c]], rows_v.at[m], gsem.at[m]
        ).wait()

    def start_scatter(c, m):
        b = b0 + lax.div(c, 2)

        @pl.when(lax.rem(m, 2) == 0)
        def _():
            pltpu.async_copy(
                rows_v.at[m], out_hbm.at[b, pl.ds(0, LANES)], ssem.at[m]
            )

        @pl.when(lax.rem(m, 2) == 1)
        def _():
            pltpu.async_copy(
                rows_v.at[m, pl.ds(0, TAIL)],
                out_hbm.at[b, pl.ds(LANES, TAIL)],
                ssem.at[m],
            )

    def wait_scatter(m):
        @pl.when(lax.rem(m, 2) == 0)
        def _():
            pltpu.make_async_copy(
                rows_v.at[m], out_hbm.at[b0, pl.ds(0, LANES)], ssem.at[m]
            ).wait()

        @pl.when(lax.rem(m, 2) == 1)
        def _():
            pltpu.make_async_copy(
                rows_v.at[m, pl.ds(0, TAIL)],
                out_hbm.at[b0, pl.ds(LANES, TAIL)],
                ssem.at[m],
            ).wait()

    start_gather(0, 0)
    start_gather(1, 1)

    def step(c, carry):
        m = lax.rem(c, NBUF)
        wait_gather(c, m)
        start_scatter(c, m)
        nc = c + 2
        mn = lax.rem(nc, NBUF)

        @pl.when(nc < NCHUNK)
        def _():
            @pl.when(nc >= NBUF)
            def _():
                wait_scatter(mn)

            start_gather(nc, mn)

        return carry

    lax.fori_loop(0, NCHUNK, step, 0)
    wait_scatter(0)
    wait_scatter(1)
    wait_scatter(2)
    wait_scatter(3)


@jax.jit
def kernel(indices, embedding_matrix):
    table = _scale_pad_table(embedding_matrix.astype(jnp.float32))
    idx = jnp.pad(indices.astype(jnp.int32), ((0, 0), (0, NPAD - N)))
    idx = idx.reshape(NW * NCHUNK, LANES)

    mesh = plsc.VectorSubcoreMesh(core_axis_name="c", subcore_axis_name="s")
    out = pl.kernel(
        _sc_body,
        out_type=jax.ShapeDtypeStruct((B, N, LANES), jnp.float32),
        mesh=mesh,
        scratch_types=[
            pltpu.VMEM_SHARED((VOCAB, LANES), jnp.float32),
            pltpu.VMEM((NCHUNK, LANES), jnp.int32),
            pltpu.VMEM((NBUF, LANES, LANES), jnp.float32),
            pltpu.SemaphoreType.DMA((NBUF,)),
            pltpu.SemaphoreType.DMA((NBUF,)),
        ],
    )(table, idx)
    return out[:, :, :D]


# speedup vs baseline: 1.6767x; 1.2230x over previous
import jax
import jax.numpy as jnp
from jax import lax
from jax.experimental import pallas as pl
from jax.experimental.pallas import tpu as pltpu
from jax.experimental.pallas import tpu_sc as plsc

VOCAB = 256
D = 64
B = 4096
N = 200

LANES = 128
TAIL = N - LANES
GB = 16
TCH = GB * TAIL // LANES

NUM_CORES = 2
NUM_SUBCORES = 16
NW = NUM_CORES * NUM_SUBCORES

NB_PER_W = B // NW
NGRP = NB_PER_W // GB
NCHUNK = NB_PER_W + NGRP * TCH
NBUF = 4

_PIECES = []
for _t in range(TCH):
    _pieces, _pos, _left = [], _t * LANES, LANES
    while _left:
        _b, _off = divmod(_pos, TAIL)
        _ln = min(TAIL - _off, _left)
        _pieces.append((_b, LANES - _left, _off, _ln))
        _pos += _ln
        _left -= _ln
    _PIECES.append(tuple(_pieces))


def _scale_body(t_ref, o_ref):
    o_ref[...] = jnp.concatenate(
        [t_ref[...] * (D ** 0.5), jnp.zeros((VOCAB, LANES - D), jnp.float32)],
        axis=1,
    )


def _scale_pad_table(table):
    return pl.pallas_call(
        _scale_body,
        out_shape=jax.ShapeDtypeStruct((VOCAB, LANES), jnp.float32),
    )(table)


def _sc_body(table_hbm, idx_hbm, out_hbm, table_v, idx_v, rows_v, gsem, ssem):
    wid = lax.axis_index("s") * NUM_CORES + lax.axis_index("c")
    sid = lax.axis_index("s")
    b0 = wid * NB_PER_W

    @pl.when(sid == 0)
    def _():
        pltpu.sync_copy(table_hbm, table_v)

    plsc.subcore_barrier()
    pltpu.sync_copy(idx_hbm.at[pl.ds(wid * NCHUNK, NCHUNK)], idx_v)

    def start_gather(c, m):
        pltpu.async_copy(table_v.at[idx_v.at[c]], rows_v.at[m], gsem.at[m])

    def wait_gather(c, m):
        pltpu.make_async_copy(
            table_v.at[idx_v.at[c]], rows_v.at[m], gsem.at[m]
        ).wait()

    def wait_scatter(m):
        pltpu.make_async_copy(
            rows_v.at[m], out_hbm.at[b0, pl.ds(0, LANES)], ssem.at[m]
        ).wait()

    def prefetch(c):
        nc = c + 2
        mn = lax.rem(nc, NBUF)

        @pl.when(nc < NCHUNK)
        def _():
            @pl.when(nc >= NBUF)
            def _():
                wait_scatter(mn)

            start_gather(nc, mn)

    start_gather(0, 0)
    start_gather(1, 1)

    def head_step(c, carry):
        m = lax.rem(c, NBUF)
        wait_gather(c, m)
        pltpu.async_copy(
            rows_v.at[m], out_hbm.at[b0 + c, pl.ds(0, LANES)], ssem.at[m]
        )
        prefetch(c)
        return carry

    lax.fori_loop(0, NB_PER_W, head_step, 0)

    def tail_group(g, carry):
        gb = b0 + g * GB
        for t in range(TCH):
            c = NB_PER_W + g * TCH + t
            m = lax.rem(c, NBUF)
            wait_gather(c, m)
            for b_i, so, no, ln in _PIECES[t]:
                pltpu.async_copy(
                    rows_v.at[m, pl.ds(so, ln)],
                    out_hbm.at[gb + b_i, pl.ds(LANES + no, ln)],
                    ssem.at[m],
                )
            prefetch(c)
        return carry

    lax.fori_loop(0, NGRP, tail_group, 0)

    wait_scatter(0)
    wait_scatter(1)
    wait_scatter(2)
    wait_scatter(3)


@jax.jit
def kernel(indices, embedding_matrix):
    table = _scale_pad_table(embedding_matrix.astype(jnp.float32))
    idx32 = indices.astype(jnp.int32)
    heads = idx32[:, :LANES].reshape(NW, NB_PER_W, LANES)
    tails = idx32[:, LANES:].reshape(NW, NGRP * TCH, LANES)
    idx = jnp.concatenate([heads, tails], axis=1).reshape(NW * NCHUNK, LANES)

    mesh = plsc.VectorSubcoreMesh(core_axis_name="c", subcore_axis_name="s")
    out = pl.kernel(
        _sc_body,
        out_type=jax.ShapeDtypeStruct((B, N, LANES), jnp.float32),
        mesh=mesh,
        scratch_types=[
            pltpu.VMEM_SHARED((VOCAB, LANES), jnp.float32),
            pltpu.VMEM((NCHUNK, LANES), jnp.int32),
            pltpu.VMEM((NBUF, LANES, LANES), jnp.float32),
            pltpu.SemaphoreType.DMA((NBUF,)),
            pltpu.SemaphoreType.DMA((NBUF,)),
        ],
    )(table, idx)
    return out[:, :, :D]


# speedup vs baseline: 1.6830x; 1.0037x over previous
import jax
import jax.numpy as jnp
from jax import lax
from jax.experimental import pallas as pl
from jax.experimental.pallas import tpu as pltpu
from jax.experimental.pallas import tpu_sc as plsc

VOCAB = 256
D = 64
B = 4096
N = 200

LANES = 128
TAIL = N - LANES
GB = 16
TCH = GB * TAIL // LANES

NUM_CORES = 2
NUM_SUBCORES = 16
NW = NUM_CORES * NUM_SUBCORES

NB_PER_W = B // NW
NGRP = NB_PER_W // GB
NCHUNK = NB_PER_W + NGRP * TCH
NBUF = 6

_PIECES = []
for _t in range(TCH):
    _pieces, _pos, _left = [], _t * LANES, LANES
    while _left:
        _b, _off = divmod(_pos, TAIL)
        _ln = min(TAIL - _off, _left)
        _pieces.append((_b, LANES - _left, _off, _ln))
        _pos += _ln
        _left -= _ln
    _PIECES.append(tuple(_pieces))


def _scale_body(t_ref, o_ref):
    o_ref[...] = jnp.concatenate(
        [t_ref[...] * (D ** 0.5), jnp.zeros((VOCAB, LANES - D), jnp.float32)],
        axis=1,
    )


def _scale_pad_table(table):
    return pl.pallas_call(
        _scale_body,
        out_shape=jax.ShapeDtypeStruct((VOCAB, LANES), jnp.float32),
    )(table)


def _sc_body(table_hbm, idx_hbm, out_hbm, table_v, idx_v, rows_v, gsem, ssem):
    wid = lax.axis_index("s") * NUM_CORES + lax.axis_index("c")
    sid = lax.axis_index("s")
    b0 = wid * NB_PER_W

    @pl.when(sid == 0)
    def _():
        pltpu.sync_copy(table_hbm, table_v)

    plsc.subcore_barrier()
    pltpu.sync_copy(idx_hbm.at[pl.ds(wid * NCHUNK, NCHUNK)], idx_v)

    def start_gather(c, m):
        pltpu.async_copy(table_v.at[idx_v.at[c]], rows_v.at[m], gsem.at[m])

    def wait_gather(c, m):
        pltpu.make_async_copy(
            table_v.at[idx_v.at[c]], rows_v.at[m], gsem.at[m]
        ).wait()

    def wait_scatter(m):
        pltpu.make_async_copy(
            rows_v.at[m], out_hbm.at[b0, pl.ds(0, LANES)], ssem.at[m]
        ).wait()

    def prefetch(c):
        nc = c + 3
        mn = lax.rem(nc, NBUF)

        @pl.when(nc < NCHUNK)
        def _():
            @pl.when(nc >= NBUF)
            def _():
                wait_scatter(mn)

            start_gather(nc, mn)

    start_gather(0, 0)
    start_gather(1, 1)
    start_gather(2, 2)

    def head_step(c, carry):
        m = lax.rem(c, NBUF)
        wait_gather(c, m)
        pltpu.async_copy(
            rows_v.at[m], out_hbm.at[b0 + c, pl.ds(0, LANES)], ssem.at[m]
        )
        prefetch(c)
        return carry

    lax.fori_loop(0, NB_PER_W, head_step, 0)

    def tail_group(g, carry):
        gb = b0 + g * GB
        for t in range(TCH):
            c = NB_PER_W + g * TCH + t
            m = lax.rem(c, NBUF)
            wait_gather(c, m)
            for b_i, so, no, ln in _PIECES[t]:
                pltpu.async_copy(
                    rows_v.at[m, pl.ds(so, ln)],
                    out_hbm.at[gb + b_i, pl.ds(LANES + no, ln)],
                    ssem.at[m],
                )
            prefetch(c)
        return carry

    lax.fori_loop(0, NGRP, tail_group, 0)

    for _m in range(NBUF):
        wait_scatter(_m)


@jax.jit
def kernel(indices, embedding_matrix):
    table = _scale_pad_table(embedding_matrix.astype(jnp.float32))
    idx32 = indices.astype(jnp.int32)
    heads = idx32[:, :LANES].reshape(NW, NB_PER_W, LANES)
    tails = idx32[:, LANES:].reshape(NW, NGRP * TCH, LANES)
    idx = jnp.concatenate([heads, tails], axis=1).reshape(NW * NCHUNK, LANES)

    mesh = plsc.VectorSubcoreMesh(core_axis_name="c", subcore_axis_name="s")
    out = pl.kernel(
        _sc_body,
        out_type=jax.ShapeDtypeStruct((B, N, LANES), jnp.float32),
        mesh=mesh,
        scratch_types=[
            pltpu.VMEM_SHARED((VOCAB, LANES), jnp.float32),
            pltpu.VMEM((NCHUNK, LANES), jnp.int32),
            pltpu.VMEM((NBUF, LANES, LANES), jnp.float32),
            pltpu.SemaphoreType.DMA((NBUF,)),
            pltpu.SemaphoreType.DMA((NBUF,)),
        ],
    )(table, idx)
    return out[:, :, :D]
